# Initial kernel scaffold; baseline (speedup 1.0000x reference)
import jax
import jax.numpy as jnp
from jax.experimental import pallas as pl


def kernel(x, conv1_w, bn1_s, bn1_b, fc_w, fc_b, L0B0_w1, L0B0_w2, L0B0_s1, L0B0_b1, L0B0_s2, L0B0_b2, L0B1_w1, L0B1_w2, L0B1_s1, L0B1_b1, L0B1_s2, L0B1_b2, L1B0_w1, L1B0_w2, L1B0_s1, L1B0_b1, L1B0_s2, L1B0_b2, L1B0_wd, L1B0_sd, L1B0_bd, L1B1_w1, L1B1_w2, L1B1_s1, L1B1_b1, L1B1_s2, L1B1_b2, L2B0_w1, L2B0_w2, L2B0_s1, L2B0_b1, L2B0_s2, L2B0_b2, L2B0_wd, L2B0_sd, L2B0_bd, L2B1_w1, L2B1_w2, L2B1_s1, L2B1_b1, L2B1_s2, L2B1_b2, L3B0_w1, L3B0_w2, L3B0_s1, L3B0_b1, L3B0_s2, L3B0_b2, L3B0_wd, L3B0_sd, L3B0_bd, L3B1_w1, L3B1_w2, L3B1_s1, L3B1_b1, L3B1_s2, L3B1_b2):
    raise NotImplementedError("write your pallas kernel here")


# speedup vs baseline: 1.0158x; 1.0158x over previous
import functools

import jax
import jax.numpy as jnp
from jax.experimental import pallas as pl
from jax.experimental.pallas import tpu as pltpu

_VMEM_LIMIT = 48 << 20


def _cparams():
    return pltpu.CompilerParams(
        dimension_semantics=("parallel",), vmem_limit_bytes=_VMEM_LIMIT
    )


def _stem_kernel(p_ref, w_ref, s_ref, b_ref, o_ref):
    acc = jnp.dot(p_ref[0], w_ref[0], preferred_element_type=jnp.float32)
    acc = jnp.maximum(acc * s_ref[...] + b_ref[...], 0.0)
    o_ref[0] = acc.astype(o_ref.dtype)


def _pool_kernel(x_ref, mask_ref, o_ref, *, offsets, m, P, Lp):
    r = x_ref[0, pl.ds(offsets[0], m), :]
    for off in offsets[1:]:
        r = jnp.maximum(r, x_ref[0, pl.ds(off, m), :])
    r = (r.astype(jnp.float32) * mask_ref[...]).astype(o_ref.dtype)
    o_ref[0] = jnp.pad(r, ((P + 1, Lp - m - P - 1), (0, 0)))


def _block_s1_kernel(
    x_ref, w1_ref, s1_ref, b1_ref, w2_ref, s2_ref, b2_ref, mask_ref, o_ref,
    scratch_ref, *, P, m, Lp
):
    offs = tuple(di * P + dj for di in range(3) for dj in range(3))
    cout = o_ref.shape[-1]
    acc = jnp.zeros((m, cout), jnp.float32)
    for t, off in enumerate(offs):
        acc = acc + jnp.dot(
            x_ref[0, pl.ds(off, m), :], w1_ref[t],
            preferred_element_type=jnp.float32,
        )
    acc = jnp.maximum(acc * s1_ref[...] + b1_ref[...], 0.0) * mask_ref[...]
    scratch_ref[...] = jnp.pad(
        acc.astype(jnp.bfloat16), ((P + 1, Lp - m - P - 1), (0, 0))
    )
    acc2 = jnp.zeros((m, cout), jnp.float32)
    for t, off in enumerate(offs):
        acc2 = acc2 + jnp.dot(
            scratch_ref[pl.ds(off, m), :], w2_ref[t],
            preferred_element_type=jnp.float32,
        )
    acc2 = acc2 * s2_ref[...] + b2_ref[...]
    acc2 = acc2 + x_ref[0, pl.ds(P + 1, m), :].astype(jnp.float32)
    acc2 = jnp.maximum(acc2, 0.0) * mask_ref[...]
    o_ref[0] = jnp.pad(
        acc2.astype(jnp.bfloat16), ((P + 1, Lp - m - P - 1), (0, 0))
    )


def _block_s2_kernel(
    x_ref, w1_ref, s1_ref, b1_ref, w2_ref, s2_ref, b2_ref,
    wd_ref, sd_ref, bd_ref, mask_ref, o_ref, scratch_ref,
    *, R, P, m, Lp
):
    offs1 = tuple(
        (2 * (di % 2) + (dj % 2)) * R * P + (di // 2) * P + (dj // 2)
        for di in range(3)
        for dj in range(3)
    )
    cout = o_ref.shape[-1]
    acc = jnp.zeros((m, cout), jnp.float32)
    for t, off in enumerate(offs1):
        acc = acc + jnp.dot(
            x_ref[0, pl.ds(off, m), :], w1_ref[t],
            preferred_element_type=jnp.float32,
        )
    acc = jnp.maximum(acc * s1_ref[...] + b1_ref[...], 0.0) * mask_ref[...]
    scratch_ref[...] = jnp.pad(
        acc.astype(jnp.bfloat16), ((P + 1, Lp - m - P - 1), (0, 0))
    )
    ds = jnp.dot(
        x_ref[0, pl.ds(3 * R * P, m), :], wd_ref[0],
        preferred_element_type=jnp.float32,
    )
    ds = ds * sd_ref[...] + bd_ref[...]
    offs2 = tuple(di * P + dj for di in range(3) for dj in range(3))
    acc2 = jnp.zeros((m, cout), jnp.float32)
    for t, off in enumerate(offs2):
        acc2 = acc2 + jnp.dot(
            scratch_ref[pl.ds(off, m), :], w2_ref[t],
            preferred_element_type=jnp.float32,
        )
    acc2 = jnp.maximum(acc2 * s2_ref[...] + b2_ref[...] + ds, 0.0) * mask_ref[...]
    o_ref[0] = jnp.pad(
        acc2.astype(jnp.bfloat16), ((P + 1, Lp - m - P - 1), (0, 0))
    )


def _head_kernel(x_ref, w_ref, b_ref, o_ref, *, hw):
    feat = jnp.sum(x_ref[...].astype(jnp.float32), axis=1) * (1.0 / hw)
    o_ref[...] = (
        jnp.dot(feat, w_ref[...], preferred_element_type=jnp.float32) + b_ref[...]
    )


def _col_mask(m, P, wo):
    return (jnp.arange(m) % P < wo).astype(jnp.float32).reshape(m, 1)


def _phase_split(xp, R, P):
    n, hp, wp, c = xp.shape
    planes = []
    for a in (0, 1):
        for b in (0, 1):
            p = xp[:, a::2, b::2, :]
            p = jnp.pad(
                p, ((0, 0), (0, R - p.shape[1]), (0, P - p.shape[2]), (0, 0))
            )
            planes.append(p.reshape(n, R * P, c))
    return jnp.concatenate(planes, axis=1)


def _padded_to_image(x_flat, h, w, c):
    n = x_flat.shape[0]
    hp, wp = h + 2, w + 2
    return x_flat[:, : hp * wp, :].reshape(n, hp, wp, c)


def _stem(x_nhwc, w, s, b):
    n, h, wd_, cin = x_nhwc.shape
    k, st, pad = 7, 2, 3
    ho = (h + 2 * pad - k) // st + 1
    wo = (wd_ + 2 * pad - k) // st + 1
    xp = jnp.pad(x_nhwc, ((0, 0), (pad, pad), (pad, pad), (0, 0)))
    cols = []
    for di in range(k):
        for dj in range(k):
            cols.append(
                jax.lax.slice(
                    xp,
                    (0, di, dj, 0),
                    (n, di + st * (ho - 1) + 1, dj + st * (wo - 1) + 1, cin),
                    (1, st, st, 1),
                )
            )
    patches = jnp.concatenate(cols, axis=-1).reshape(n, ho * wo, k * k * cin)
    kk = k * k * cin
    cout = w.shape[-1]
    y = pl.pallas_call(
        _stem_kernel,
        grid=(n,),
        in_specs=[
            pl.BlockSpec((1, ho * wo, kk), lambda i: (i, 0, 0)),
            pl.BlockSpec((1, kk, cout), lambda i: (0, 0, 0)),
            pl.BlockSpec((1, cout), lambda i: (0, 0)),
            pl.BlockSpec((1, cout), lambda i: (0, 0)),
        ],
        out_shape=jax.ShapeDtypeStruct((n, ho * wo, cout), jnp.bfloat16),
        out_specs=pl.BlockSpec((1, ho * wo, cout), lambda i: (i, 0, 0)),
        compiler_params=_cparams(),
    )(patches, w, s, b)
    return y.reshape(n, ho, wo, cout)


def _maxpool(x):
    n, h, w, c = x.shape
    ho, wo = h // 2, w // 2
    R = P = wo + 2
    xp = jnp.pad(x, ((0, 0), (1, 1), (1, 1), (0, 0)))
    flat = _phase_split(xp, R, P)
    m = ho * P
    Lp = (ho + 3) * P
    offs = tuple(
        (2 * (di % 2) + (dj % 2)) * R * P + (di // 2) * P + (dj // 2)
        for di in range(3)
        for dj in range(3)
    )
    mask = _col_mask(m, P, wo)
    return pl.pallas_call(
        functools.partial(_pool_kernel, offsets=offs, m=m, P=P, Lp=Lp),
        grid=(n,),
        in_specs=[
            pl.BlockSpec((1, 4 * R * P, c), lambda i: (i, 0, 0)),
            pl.BlockSpec((m, 1), lambda i: (0, 0)),
        ],
        out_shape=jax.ShapeDtypeStruct((n, Lp, c), jnp.bfloat16),
        out_specs=pl.BlockSpec((1, Lp, c), lambda i: (i, 0, 0)),
        compiler_params=_cparams(),
    )(flat, mask)


def _block_s1(x_flat, h, w, w1, s1, b1, w2, s2, b2):
    n, Lp, c = x_flat.shape
    P = w + 2
    m = h * P
    cout = w1.shape[-1]
    mask = _col_mask(m, P, w)
    return pl.pallas_call(
        functools.partial(_block_s1_kernel, P=P, m=m, Lp=Lp),
        grid=(n,),
        in_specs=[
            pl.BlockSpec((1, Lp, c), lambda i: (i, 0, 0)),
            pl.BlockSpec((9, c, cout), lambda i: (0, 0, 0)),
            pl.BlockSpec((1, cout), lambda i: (0, 0)),
            pl.BlockSpec((1, cout), lambda i: (0, 0)),
            pl.BlockSpec((9, cout, cout), lambda i: (0, 0, 0)),
            pl.BlockSpec((1, cout), lambda i: (0, 0)),
            pl.BlockSpec((1, cout), lambda i: (0, 0)),
            pl.BlockSpec((m, 1), lambda i: (0, 0)),
        ],
        out_shape=jax.ShapeDtypeStruct((n, Lp, cout), jnp.bfloat16),
        out_specs=pl.BlockSpec((1, Lp, cout), lambda i: (i, 0, 0)),
        scratch_shapes=[pltpu.VMEM((Lp, cout), jnp.bfloat16)],
        compiler_params=_cparams(),
    )(x_flat, w1, s1, b1, w2, s2, b2, mask)


def _block_s2(x_flat, h, w, w1, s1, b1, w2, s2, b2, wdn, sd, bd):
    n, _, cin = x_flat.shape
    ho, wo = h // 2, w // 2
    P = wo + 2
    R = ho + 2
    xp = _padded_to_image(x_flat, h, w, cin)
    flat = _phase_split(xp, R, P)
    m = ho * P
    Lp = (ho + 3) * P
    cout = w1.shape[-1]
    mask = _col_mask(m, P, wo)
    return pl.pallas_call(
        functools.partial(_block_s2_kernel, R=R, P=P, m=m, Lp=Lp),
        grid=(n,),
        in_specs=[
            pl.BlockSpec((1, 4 * R * P, cin), lambda i: (i, 0, 0)),
            pl.BlockSpec((9, cin, cout), lambda i: (0, 0, 0)),
            pl.BlockSpec((1, cout), lambda i: (0, 0)),
            pl.BlockSpec((1, cout), lambda i: (0, 0)),
            pl.BlockSpec((9, cout, cout), lambda i: (0, 0, 0)),
            pl.BlockSpec((1, cout), lambda i: (0, 0)),
            pl.BlockSpec((1, cout), lambda i: (0, 0)),
            pl.BlockSpec((1, cin, cout), lambda i: (0, 0, 0)),
            pl.BlockSpec((1, cout), lambda i: (0, 0)),
            pl.BlockSpec((1, cout), lambda i: (0, 0)),
            pl.BlockSpec((m, 1), lambda i: (0, 0)),
        ],
        out_shape=jax.ShapeDtypeStruct((n, Lp, cout), jnp.bfloat16),
        out_specs=pl.BlockSpec((1, Lp, cout), lambda i: (i, 0, 0)),
        scratch_shapes=[pltpu.VMEM((Lp, cout), jnp.bfloat16)],
        compiler_params=_cparams(),
    )(flat, w1, s1, b1, w2, s2, b2, wdn, sd, bd, mask)


def _head(x_flat, hw, fc_w, fc_b):
    n, Lp, c = x_flat.shape
    return pl.pallas_call(
        functools.partial(_head_kernel, hw=hw),
        out_shape=jax.ShapeDtypeStruct((n, fc_w.shape[1]), jnp.float32),
        compiler_params=pltpu.CompilerParams(vmem_limit_bytes=_VMEM_LIMIT),
    )(x_flat, fc_w, fc_b)


def kernel(x, conv1_w, bn1_s, bn1_b, fc_w, fc_b, L0B0_w1, L0B0_w2, L0B0_s1, L0B0_b1, L0B0_s2, L0B0_b2, L0B1_w1, L0B1_w2, L0B1_s1, L0B1_b1, L0B1_s2, L0B1_b2, L1B0_w1, L1B0_w2, L1B0_s1, L1B0_b1, L1B0_s2, L1B0_b2, L1B0_wd, L1B0_sd, L1B0_bd, L1B1_w1, L1B1_w2, L1B1_s1, L1B1_b1, L1B1_s2, L1B1_b2, L2B0_w1, L2B0_w2, L2B0_s1, L2B0_b1, L2B0_s2, L2B0_b2, L2B0_wd, L2B0_sd, L2B0_bd, L2B1_w1, L2B1_w2, L2B1_s1, L2B1_b1, L2B1_s2, L2B1_b2, L3B0_w1, L3B0_w2, L3B0_s1, L3B0_b1, L3B0_s2, L3B0_b2, L3B0_wd, L3B0_sd, L3B0_bd, L3B1_w1, L3B1_w2, L3B1_s1, L3B1_b1, L3B1_s2, L3B1_b2):
    xh = jnp.transpose(x, (0, 2, 3, 1)).astype(jnp.bfloat16)
    y = _stem(xh, conv1_w, bn1_s, bn1_b)
    f = _maxpool(y)
    f = _block_s1(f, 56, 56, L0B0_w1, L0B0_s1, L0B0_b1, L0B0_w2, L0B0_s2, L0B0_b2)
    f = _block_s1(f, 56, 56, L0B1_w1, L0B1_s1, L0B1_b1, L0B1_w2, L0B1_s2, L0B1_b2)
    f = _block_s2(f, 56, 56, L1B0_w1, L1B0_s1, L1B0_b1, L1B0_w2, L1B0_s2, L1B0_b2,
                  L1B0_wd, L1B0_sd, L1B0_bd)
    f = _block_s1(f, 28, 28, L1B1_w1, L1B1_s1, L1B1_b1, L1B1_w2, L1B1_s2, L1B1_b2)
    f = _block_s2(f, 28, 28, L2B0_w1, L2B0_s1, L2B0_b1, L2B0_w2, L2B0_s2, L2B0_b2,
                  L2B0_wd, L2B0_sd, L2B0_bd)
    f = _block_s1(f, 14, 14, L2B1_w1, L2B1_s1, L2B1_b1, L2B1_w2, L2B1_s2, L2B1_b2)
    f = _block_s2(f, 14, 14, L3B0_w1, L3B0_s1, L3B0_b1, L3B0_w2, L3B0_s2, L3B0_b2,
                  L3B0_wd, L3B0_sd, L3B0_bd)
    f = _block_s1(f, 7, 7, L3B1_w1, L3B1_s1, L3B1_b1, L3B1_w2, L3B1_s2, L3B1_b2)
    return _head(f, 49.0, fc_w, fc_b)


# speedup vs baseline: 1.0788x; 1.0620x over previous
import functools

import jax
import jax.numpy as jnp
from jax.experimental import pallas as pl
from jax.experimental.pallas import tpu as pltpu

_VMEM_LIMIT = 48 << 20


def _cparams():
    return pltpu.CompilerParams(
        dimension_semantics=("parallel",), vmem_limit_bytes=_VMEM_LIMIT
    )


def _stem_kernel(p_ref, w_ref, s_ref, b_ref, o_ref):
    acc = jnp.dot(p_ref[0], w_ref[0], preferred_element_type=jnp.float32)
    acc = jnp.maximum(acc * s_ref[...] + b_ref[...], 0.0)
    o_ref[0] = acc.astype(o_ref.dtype)


def _pool_kernel(x_ref, mask_ref, o_ref, *, offsets, m, P, Lp):
    r = x_ref[0, pl.ds(offsets[0], m), :]
    for off in offsets[1:]:
        r = jnp.maximum(r, x_ref[0, pl.ds(off, m), :])
    r = (r.astype(jnp.float32) * mask_ref[...]).astype(o_ref.dtype)
    o_ref[0] = jnp.pad(r, ((P + 1, Lp - m - P - 1), (0, 0)))


def _block_s1_kernel(
    x_ref, w1_ref, s1_ref, b1_ref, w2_ref, s2_ref, b2_ref, mask_ref, o_ref,
    scratch_ref, *, P, m, Lp
):
    offs = tuple(di * P + dj for di in range(3) for dj in range(3))
    xcat = jnp.concatenate([x_ref[0, pl.ds(off, m), :] for off in offs], axis=1)
    acc = jnp.dot(xcat, w1_ref[0], preferred_element_type=jnp.float32)
    acc = jnp.maximum(acc * s1_ref[...] + b1_ref[...], 0.0) * mask_ref[...]
    scratch_ref[...] = jnp.pad(
        acc.astype(jnp.bfloat16), ((P + 1, Lp - m - P - 1), (0, 0))
    )
    scat = jnp.concatenate(
        [scratch_ref[pl.ds(off, m), :] for off in offs], axis=1
    )
    acc2 = jnp.dot(scat, w2_ref[0], preferred_element_type=jnp.float32)
    acc2 = acc2 * s2_ref[...] + b2_ref[...]
    acc2 = acc2 + x_ref[0, pl.ds(P + 1, m), :].astype(jnp.float32)
    acc2 = jnp.maximum(acc2, 0.0) * mask_ref[...]
    o_ref[0] = jnp.pad(
        acc2.astype(jnp.bfloat16), ((P + 1, Lp - m - P - 1), (0, 0))
    )


def _block_s2_kernel(
    x_ref, w1_ref, s1_ref, b1_ref, w2_ref, s2_ref, b2_ref,
    wd_ref, sd_ref, bd_ref, mask_ref, o_ref, scratch_ref,
    *, R, P, m, Lp
):
    offs1 = tuple(
        (2 * (di % 2) + (dj % 2)) * R * P + (di // 2) * P + (dj // 2)
        for di in range(3)
        for dj in range(3)
    )
    xcat = jnp.concatenate([x_ref[0, pl.ds(off, m), :] for off in offs1], axis=1)
    acc = jnp.dot(xcat, w1_ref[0], preferred_element_type=jnp.float32)
    acc = jnp.maximum(acc * s1_ref[...] + b1_ref[...], 0.0) * mask_ref[...]
    scratch_ref[...] = jnp.pad(
        acc.astype(jnp.bfloat16), ((P + 1, Lp - m - P - 1), (0, 0))
    )
    ds = jnp.dot(
        x_ref[0, pl.ds(3 * R * P, m), :], wd_ref[0],
        preferred_element_type=jnp.float32,
    )
    ds = ds * sd_ref[...] + bd_ref[...]
    offs2 = tuple(di * P + dj for di in range(3) for dj in range(3))
    scat = jnp.concatenate(
        [scratch_ref[pl.ds(off, m), :] for off in offs2], axis=1
    )
    acc2 = jnp.dot(scat, w2_ref[0], preferred_element_type=jnp.float32)
    acc2 = jnp.maximum(acc2 * s2_ref[...] + b2_ref[...] + ds, 0.0) * mask_ref[...]
    o_ref[0] = jnp.pad(
        acc2.astype(jnp.bfloat16), ((P + 1, Lp - m - P - 1), (0, 0))
    )


def _head_kernel(x_ref, w_ref, b_ref, o_ref, *, hw):
    feat = jnp.sum(x_ref[...].astype(jnp.float32), axis=1) * (1.0 / hw)
    o_ref[...] = (
        jnp.dot(feat, w_ref[...], preferred_element_type=jnp.float32) + b_ref[...]
    )


def _col_mask(m, P, wo):
    return (jnp.arange(m) % P < wo).astype(jnp.float32).reshape(m, 1)


def _phase_split(xp, R, P):
    n, hp, wp, c = xp.shape
    planes = []
    for a in (0, 1):
        for b in (0, 1):
            p = xp[:, a::2, b::2, :]
            p = jnp.pad(
                p, ((0, 0), (0, R - p.shape[1]), (0, P - p.shape[2]), (0, 0))
            )
            planes.append(p.reshape(n, R * P, c))
    return jnp.concatenate(planes, axis=1)


def _padded_to_image(x_flat, h, w, c):
    n = x_flat.shape[0]
    hp, wp = h + 2, w + 2
    return x_flat[:, : hp * wp, :].reshape(n, hp, wp, c)


def _stem(x_nhwc, w, s, b):
    n, h, wd_, cin = x_nhwc.shape
    k, st, pad = 7, 2, 3
    ho = (h + 2 * pad - k) // st + 1
    wo = (wd_ + 2 * pad - k) // st + 1
    xp = jnp.pad(x_nhwc, ((0, 0), (pad, pad), (pad, pad), (0, 0)))
    cols = []
    for di in range(k):
        for dj in range(k):
            cols.append(
                jax.lax.slice(
                    xp,
                    (0, di, dj, 0),
                    (n, di + st * (ho - 1) + 1, dj + st * (wo - 1) + 1, cin),
                    (1, st, st, 1),
                )
            )
    patches = jnp.concatenate(cols, axis=-1).reshape(n, ho * wo, k * k * cin)
    kk = k * k * cin
    cout = w.shape[-1]
    y = pl.pallas_call(
        _stem_kernel,
        grid=(n,),
        in_specs=[
            pl.BlockSpec((1, ho * wo, kk), lambda i: (i, 0, 0)),
            pl.BlockSpec((1, kk, cout), lambda i: (0, 0, 0)),
            pl.BlockSpec((1, cout), lambda i: (0, 0)),
            pl.BlockSpec((1, cout), lambda i: (0, 0)),
        ],
        out_shape=jax.ShapeDtypeStruct((n, ho * wo, cout), jnp.bfloat16),
        out_specs=pl.BlockSpec((1, ho * wo, cout), lambda i: (i, 0, 0)),
        compiler_params=_cparams(),
    )(patches, w, s, b)
    return y.reshape(n, ho, wo, cout)


def _maxpool(x):
    n, h, w, c = x.shape
    ho, wo = h // 2, w // 2
    R = P = wo + 2
    xp = jnp.pad(x, ((0, 0), (1, 1), (1, 1), (0, 0)))
    flat = _phase_split(xp, R, P)
    m = ho * P
    Lp = (ho + 3) * P
    offs = tuple(
        (2 * (di % 2) + (dj % 2)) * R * P + (di // 2) * P + (dj // 2)
        for di in range(3)
        for dj in range(3)
    )
    mask = _col_mask(m, P, wo)
    return pl.pallas_call(
        functools.partial(_pool_kernel, offsets=offs, m=m, P=P, Lp=Lp),
        grid=(n,),
        in_specs=[
            pl.BlockSpec((1, 4 * R * P, c), lambda i: (i, 0, 0)),
            pl.BlockSpec((m, 1), lambda i: (0, 0)),
        ],
        out_shape=jax.ShapeDtypeStruct((n, Lp, c), jnp.bfloat16),
        out_specs=pl.BlockSpec((1, Lp, c), lambda i: (i, 0, 0)),
        compiler_params=_cparams(),
    )(flat, mask)


def _block_s1(x_flat, h, w, w1, s1, b1, w2, s2, b2):
    n, Lp, c = x_flat.shape
    P = w + 2
    m = h * P
    cout = w1.shape[-1]
    w1 = w1.reshape(1, 9 * c, cout)
    w2 = w2.reshape(1, 9 * cout, cout)
    mask = _col_mask(m, P, w)
    return pl.pallas_call(
        functools.partial(_block_s1_kernel, P=P, m=m, Lp=Lp),
        grid=(n,),
        in_specs=[
            pl.BlockSpec((1, Lp, c), lambda i: (i, 0, 0)),
            pl.BlockSpec((1, 9 * c, cout), lambda i: (0, 0, 0)),
            pl.BlockSpec((1, cout), lambda i: (0, 0)),
            pl.BlockSpec((1, cout), lambda i: (0, 0)),
            pl.BlockSpec((1, 9 * cout, cout), lambda i: (0, 0, 0)),
            pl.BlockSpec((1, cout), lambda i: (0, 0)),
            pl.BlockSpec((1, cout), lambda i: (0, 0)),
            pl.BlockSpec((m, 1), lambda i: (0, 0)),
        ],
        out_shape=jax.ShapeDtypeStruct((n, Lp, cout), jnp.bfloat16),
        out_specs=pl.BlockSpec((1, Lp, cout), lambda i: (i, 0, 0)),
        scratch_shapes=[pltpu.VMEM((Lp, cout), jnp.bfloat16)],
        compiler_params=_cparams(),
    )(x_flat, w1, s1, b1, w2, s2, b2, mask)


def _block_s2(x_flat, h, w, w1, s1, b1, w2, s2, b2, wdn, sd, bd):
    n, _, cin = x_flat.shape
    ho, wo = h // 2, w // 2
    P = wo + 2
    R = ho + 2
    xp = _padded_to_image(x_flat, h, w, cin)
    flat = _phase_split(xp, R, P)
    m = ho * P
    Lp = (ho + 3) * P
    cout = w1.shape[-1]
    w1 = w1.reshape(1, 9 * cin, cout)
    w2 = w2.reshape(1, 9 * cout, cout)
    mask = _col_mask(m, P, wo)
    return pl.pallas_call(
        functools.partial(_block_s2_kernel, R=R, P=P, m=m, Lp=Lp),
        grid=(n,),
        in_specs=[
            pl.BlockSpec((1, 4 * R * P, cin), lambda i: (i, 0, 0)),
            pl.BlockSpec((1, 9 * cin, cout), lambda i: (0, 0, 0)),
            pl.BlockSpec((1, cout), lambda i: (0, 0)),
            pl.BlockSpec((1, cout), lambda i: (0, 0)),
            pl.BlockSpec((1, 9 * cout, cout), lambda i: (0, 0, 0)),
            pl.BlockSpec((1, cout), lambda i: (0, 0)),
            pl.BlockSpec((1, cout), lambda i: (0, 0)),
            pl.BlockSpec((1, cin, cout), lambda i: (0, 0, 0)),
            pl.BlockSpec((1, cout), lambda i: (0, 0)),
            pl.BlockSpec((1, cout), lambda i: (0, 0)),
            pl.BlockSpec((m, 1), lambda i: (0, 0)),
        ],
        out_shape=jax.ShapeDtypeStruct((n, Lp, cout), jnp.bfloat16),
        out_specs=pl.BlockSpec((1, Lp, cout), lambda i: (i, 0, 0)),
        scratch_shapes=[pltpu.VMEM((Lp, cout), jnp.bfloat16)],
        compiler_params=_cparams(),
    )(flat, w1, s1, b1, w2, s2, b2, wdn, sd, bd, mask)


def _head(x_flat, hw, fc_w, fc_b):
    n, Lp, c = x_flat.shape
    return pl.pallas_call(
        functools.partial(_head_kernel, hw=hw),
        out_shape=jax.ShapeDtypeStruct((n, fc_w.shape[1]), jnp.float32),
        compiler_params=pltpu.CompilerParams(vmem_limit_bytes=_VMEM_LIMIT),
    )(x_flat, fc_w, fc_b)


def kernel(x, conv1_w, bn1_s, bn1_b, fc_w, fc_b, L0B0_w1, L0B0_w2, L0B0_s1, L0B0_b1, L0B0_s2, L0B0_b2, L0B1_w1, L0B1_w2, L0B1_s1, L0B1_b1, L0B1_s2, L0B1_b2, L1B0_w1, L1B0_w2, L1B0_s1, L1B0_b1, L1B0_s2, L1B0_b2, L1B0_wd, L1B0_sd, L1B0_bd, L1B1_w1, L1B1_w2, L1B1_s1, L1B1_b1, L1B1_s2, L1B1_b2, L2B0_w1, L2B0_w2, L2B0_s1, L2B0_b1, L2B0_s2, L2B0_b2, L2B0_wd, L2B0_sd, L2B0_bd, L2B1_w1, L2B1_w2, L2B1_s1, L2B1_b1, L2B1_s2, L2B1_b2, L3B0_w1, L3B0_w2, L3B0_s1, L3B0_b1, L3B0_s2, L3B0_b2, L3B0_wd, L3B0_sd, L3B0_bd, L3B1_w1, L3B1_w2, L3B1_s1, L3B1_b1, L3B1_s2, L3B1_b2):
    xh = jnp.transpose(x, (0, 2, 3, 1)).astype(jnp.bfloat16)
    y = _stem(xh, conv1_w, bn1_s, bn1_b)
    f = _maxpool(y)
    f = _block_s1(f, 56, 56, L0B0_w1, L0B0_s1, L0B0_b1, L0B0_w2, L0B0_s2, L0B0_b2)
    f = _block_s1(f, 56, 56, L0B1_w1, L0B1_s1, L0B1_b1, L0B1_w2, L0B1_s2, L0B1_b2)
    f = _block_s2(f, 56, 56, L1B0_w1, L1B0_s1, L1B0_b1, L1B0_w2, L1B0_s2, L1B0_b2,
                  L1B0_wd, L1B0_sd, L1B0_bd)
    f = _block_s1(f, 28, 28, L1B1_w1, L1B1_s1, L1B1_b1, L1B1_w2, L1B1_s2, L1B1_b2)
    f = _block_s2(f, 28, 28, L2B0_w1, L2B0_s1, L2B0_b1, L2B0_w2, L2B0_s2, L2B0_b2,
                  L2B0_wd, L2B0_sd, L2B0_bd)
    f = _block_s1(f, 14, 14, L2B1_w1, L2B1_s1, L2B1_b1, L2B1_w2, L2B1_s2, L2B1_b2)
    f = _block_s2(f, 14, 14, L3B0_w1, L3B0_s1, L3B0_b1, L3B0_w2, L3B0_s2, L3B0_b2,
                  L3B0_wd, L3B0_sd, L3B0_bd)
    f = _block_s1(f, 7, 7, L3B1_w1, L3B1_s1, L3B1_b1, L3B1_w2, L3B1_s2, L3B1_b2)
    return _head(f, 49.0, fc_w, fc_b)
